# baseline (speedup 1.0000x reference)
import functools

import jax
import jax.numpy as jnp
from jax import lax
from jax.experimental import pallas as pl
from jax.experimental.pallas import tpu as pltpu
from jax.experimental.pallas import tpu_sc as plsc

B, D, S = 4, 1024, 8192
NW = 32
N_DG = 8
D_PER_W = D // N_DG
N_SR = NW // N_DG
S_PER_W = S // N_SR
S_CHUNK = 128
N_CHUNKS = S_PER_W // S_CHUNK
L = 16


def _sc_body(x_hbm, pe_hbm, out_hbm, xt, pet):
    wid = lax.axis_index("s") * 2 + lax.axis_index("c")
    d0 = (wid % N_DG) * D_PER_W
    s_base = (wid // N_DG) * S_PER_W
    iota = lax.iota(jnp.int32, L)

    def chunk_body(c, carry):
        s0 = s_base + c * S_CHUNK
        pltpu.sync_copy(pe_hbm.at[pl.ds(s0, S_CHUNK), pl.ds(d0, D_PER_W)], pet)
        for b in range(B):
            pltpu.sync_copy(x_hbm.at[b, pl.ds(d0, D_PER_W), pl.ds(s0, S_CHUNK)],
                            xt.at[b])

        def d_body(d, carry2):
            d_idx = jnp.zeros((L,), jnp.int32) + d
            for sj in range(S_CHUNK // L):
                s_idx = sj * L + iota
                pv = plsc.load_gather(pet, [s_idx, d_idx])
                for b in range(B):
                    plsc.addupdate(xt.at[b, d, pl.ds(sj * L, L)], pv)
            return carry2

        lax.fori_loop(0, D_PER_W, d_body, 0)
        for b in range(B):
            pltpu.sync_copy(xt.at[b],
                            out_hbm.at[b, pl.ds(d0, D_PER_W), pl.ds(s0, S_CHUNK)])
        return carry

    lax.fori_loop(0, N_CHUNKS, chunk_body, 0)


def kernel(x, pe_table):
    mesh = plsc.VectorSubcoreMesh(core_axis_name="c", subcore_axis_name="s")
    k = functools.partial(
        pl.kernel,
        mesh=mesh,
        out_type=jax.ShapeDtypeStruct((B, D, S), jnp.float32),
        scratch_types=[
            pltpu.VMEM((B, D_PER_W, S_CHUNK), jnp.float32),
            pltpu.VMEM((S_CHUNK, D_PER_W), jnp.float32),
        ],
        compiler_params=pltpu.CompilerParams(needs_layout_passes=False),
    )(_sc_body)
    return k(x, pe_table)
